# baseline (speedup 1.0000x reference)
import functools

import jax
import jax.numpy as jnp
from jax import lax
from jax.experimental import pallas as pl
from jax.experimental.pallas import tpu as pltpu
from jax.experimental.pallas import tpu_sc as plsc

N_NODES = 10000
N_EDGES = 320000
FEATS = 128

NC = 2
NS = 16
NW = NC * NS
CHUNK = 128
NB = 2 * (-(-N_EDGES // (NW * CHUNK * 2)))
NBP = NB + 2
E_PAD = NW * NBP * CHUNK
NPT = (-(-N_NODES // NS) + 7) // 8 * 8
N_PAD = NPT * NS


def _scatter_body(src_hbm, dst_hbm, x_hbm, zeros_hbm, out_hbm,
                  src0, src1, dst0, dst1, rows0, rows1, acc_s,
                  sg0, sg1, ss0, ss1, sd0, sd1):
    cid = lax.axis_index("c")
    sid = lax.axis_index("s")
    wid = cid * NS + sid

    pltpu.sync_copy(zeros_hbm, acc_s.at[pl.ds(sid * NPT, NPT)])

    slots = ((src0, dst0, rows0, sg0, ss0, sd0),
             (src1, dst1, rows1, sg1, ss1, sd1))

    def idx_start(c, slot):
        src_v, dst_v, _, _, ss, sd = slot
        pltpu.async_copy(src_hbm.at[wid, c], src_v, ss)
        pltpu.async_copy(dst_hbm.at[wid, c], dst_v, sd)

    def idx_wait(c, slot):
        src_v, dst_v, _, _, ss, sd = slot
        pltpu.make_async_copy(src_hbm.at[wid, c], src_v, ss).wait()
        pltpu.make_async_copy(dst_hbm.at[wid, c], dst_v, sd).wait()

    def g_start(slot):
        src_v, _, rows_v, sg, _, _ = slot
        pltpu.async_copy(x_hbm.at[src_v], rows_v, sg)

    def g_wait(slot):
        src_v, _, rows_v, sg, _, _ = slot
        pltpu.make_async_copy(x_hbm.at[src_v], rows_v, sg).wait()

    idx_start(0, slots[0])
    idx_start(1, slots[1])
    idx_wait(0, slots[0])
    g_start(slots[0])
    plsc.subcore_barrier()

    def phase(c, cur, nxt):
        idx_wait(c + 1, nxt)
        g_wait(cur)
        g_start(nxt)
        _, dst_v, rows_v, _, _, _ = cur
        pltpu.sync_copy(rows_v, acc_s.at[dst_v], add=True)
        idx_start(c + 2, cur)

    def body(k, carry):
        c = 2 * k
        phase(c, slots[0], slots[1])
        phase(c + 1, slots[1], slots[0])
        return carry

    lax.fori_loop(0, NB // 2, body, 0)

    g_wait(slots[0])
    idx_wait(NB + 1, slots[1])
    plsc.subcore_barrier()

    pltpu.sync_copy(acc_s.at[pl.ds(sid * NPT, NPT)],
                    out_hbm.at[cid, pl.ds(sid * NPT, NPT)])


_scatter_sc = functools.partial(
    pl.kernel,
    mesh=plsc.VectorSubcoreMesh(core_axis_name="c", subcore_axis_name="s"),
    out_type=jax.ShapeDtypeStruct((NC, N_PAD, FEATS), jnp.float32),
    scratch_types=[
        pltpu.VMEM((CHUNK,), jnp.int32),
        pltpu.VMEM((CHUNK,), jnp.int32),
        pltpu.VMEM((CHUNK,), jnp.int32),
        pltpu.VMEM((CHUNK,), jnp.int32),
        pltpu.VMEM((CHUNK, FEATS), jnp.float32),
        pltpu.VMEM((CHUNK, FEATS), jnp.float32),
        pltpu.VMEM_SHARED((N_PAD, FEATS), jnp.float32),
        pltpu.SemaphoreType.DMA,
        pltpu.SemaphoreType.DMA,
        pltpu.SemaphoreType.DMA,
        pltpu.SemaphoreType.DMA,
        pltpu.SemaphoreType.DMA,
        pltpu.SemaphoreType.DMA,
    ],
)(_scatter_body)


def _linear_body(p0_ref, p1_ref, wt_ref, b_ref, o_ref):
    h = p0_ref[...] + p1_ref[...]
    o_ref[...] = (
        jnp.dot(h, wt_ref[...], preferred_element_type=jnp.float32) + b_ref[...]
    )


def _linear_tc(p0, p1, wt, b2):
    m = p0.shape[0]
    bm = 1000
    return pl.pallas_call(
        _linear_body,
        grid=(m // bm,),
        in_specs=[
            pl.BlockSpec((bm, FEATS), lambda i: (i, 0)),
            pl.BlockSpec((bm, FEATS), lambda i: (i, 0)),
            pl.BlockSpec((FEATS, FEATS), lambda i: (0, 0)),
            pl.BlockSpec((1, FEATS), lambda i: (0, 0)),
        ],
        out_specs=pl.BlockSpec((bm, FEATS), lambda i: (i, 0)),
        out_shape=jax.ShapeDtypeStruct((m, FEATS), jnp.float32),
    )(p0, p1, wt, b2)


def kernel(x, edge_index, W, b):
    src = edge_index[0].astype(jnp.int32)
    dst = edge_index[1].astype(jnp.int32)
    e_tile = NB * CHUNK
    src2 = jnp.zeros((NW, NBP * CHUNK), jnp.int32)
    dst2 = jnp.full((NW, NBP * CHUNK), N_NODES, jnp.int32)
    src2 = src2.at[:, :e_tile].set(
        jnp.concatenate([src, jnp.zeros((NW * e_tile - N_EDGES,), jnp.int32)])
        .reshape(NW, e_tile))
    dst2 = dst2.at[:, :e_tile].set(
        jnp.concatenate([dst, jnp.full((NW * e_tile - N_EDGES,), N_NODES,
                                       jnp.int32)]).reshape(NW, e_tile))
    src3 = src2.reshape(NW, NBP, CHUNK)
    dst3 = dst2.reshape(NW, NBP, CHUNK)
    zeros = jnp.zeros((NPT, FEATS), jnp.float32)
    partial = _scatter_sc(src3, dst3, x, zeros)
    return _linear_tc(partial[0, :N_NODES], partial[1, :N_NODES],
                      W.T, b.reshape(1, FEATS))


# speedup vs baseline: 1.1590x; 1.1590x over previous
import functools

import jax
import jax.numpy as jnp
from jax import lax
from jax.experimental import pallas as pl
from jax.experimental.pallas import tpu as pltpu
from jax.experimental.pallas import tpu_sc as plsc

N_NODES = 10000
N_EDGES = 320000
FEATS = 128

NC = 2
NS = 16
NW = NC * NS
CHUNK = 128
NB = 2 * (-(-N_EDGES // (NW * CHUNK * 2)))
NBP = NB + 2
E_PAD = NW * NBP * CHUNK
NPT = (-(-N_NODES // NS) + 7) // 8 * 8
N_PAD = NPT * NS


def _scatter_body(src_hbm, dst_hbm, x_hbm, zeros_hbm, out_hbm,
                  src0, src1, dst0, dst1, rows_v, acc_s,
                  sg, ss0, ss1, sd0, sd1):
    cid = lax.axis_index("c")
    sid = lax.axis_index("s")
    wid = cid * NS + sid

    pltpu.sync_copy(zeros_hbm, acc_s.at[pl.ds(sid * NPT, NPT)])

    slots = ((src0, dst0, ss0, sd0), (src1, dst1, ss1, sd1))

    def idx_start(c, slot):
        src_v, dst_v, ss, sd = slot
        pltpu.async_copy(src_hbm.at[wid, c], src_v, ss)
        pltpu.async_copy(dst_hbm.at[wid, c], dst_v, sd)

    def idx_wait(c, slot):
        src_v, dst_v, ss, sd = slot
        pltpu.make_async_copy(src_hbm.at[wid, c], src_v, ss).wait()
        pltpu.make_async_copy(dst_hbm.at[wid, c], dst_v, sd).wait()

    idx_start(0, slots[0])
    plsc.subcore_barrier()

    def phase(c, cur, nxt):
        idx_wait(c, cur)
        src_v, dst_v, _, _ = cur
        pltpu.async_copy(x_hbm.at[src_v], rows_v, sg).wait()
        idx_start(c + 1, nxt)
        pltpu.sync_copy(rows_v, acc_s.at[dst_v], add=True)

    def body(k, carry):
        c = 2 * k
        phase(c, slots[0], slots[1])
        phase(c + 1, slots[1], slots[0])
        return carry

    lax.fori_loop(0, NB // 2, body, 0)

    idx_wait(NB, slots[0])
    plsc.subcore_barrier()

    pltpu.sync_copy(acc_s.at[pl.ds(sid * NPT, NPT)],
                    out_hbm.at[cid, pl.ds(sid * NPT, NPT)])


_scatter_sc = functools.partial(
    pl.kernel,
    mesh=plsc.VectorSubcoreMesh(core_axis_name="c", subcore_axis_name="s"),
    out_type=jax.ShapeDtypeStruct((NC, N_PAD, FEATS), jnp.float32),
    scratch_types=[
        pltpu.VMEM((CHUNK,), jnp.int32),
        pltpu.VMEM((CHUNK,), jnp.int32),
        pltpu.VMEM((CHUNK,), jnp.int32),
        pltpu.VMEM((CHUNK,), jnp.int32),
        pltpu.VMEM((CHUNK, FEATS), jnp.float32),
        pltpu.VMEM_SHARED((N_PAD, FEATS), jnp.float32),
        pltpu.SemaphoreType.DMA,
        pltpu.SemaphoreType.DMA,
        pltpu.SemaphoreType.DMA,
        pltpu.SemaphoreType.DMA,
        pltpu.SemaphoreType.DMA,
    ],
)(_scatter_body)


def _linear_body(p0_ref, p1_ref, wt_ref, b_ref, o_ref):
    h = p0_ref[...] + p1_ref[...]
    o_ref[...] = (
        jnp.dot(h, wt_ref[...], preferred_element_type=jnp.float32) + b_ref[...]
    )


def _linear_tc(p0, p1, wt, b2):
    m = p0.shape[0]
    bm = 1000
    return pl.pallas_call(
        _linear_body,
        grid=(m // bm,),
        in_specs=[
            pl.BlockSpec((bm, FEATS), lambda i: (i, 0)),
            pl.BlockSpec((bm, FEATS), lambda i: (i, 0)),
            pl.BlockSpec((FEATS, FEATS), lambda i: (0, 0)),
            pl.BlockSpec((1, FEATS), lambda i: (0, 0)),
        ],
        out_specs=pl.BlockSpec((bm, FEATS), lambda i: (i, 0)),
        out_shape=jax.ShapeDtypeStruct((m, FEATS), jnp.float32),
    )(p0, p1, wt, b2)


def kernel(x, edge_index, W, b):
    src = edge_index[0].astype(jnp.int32)
    dst = edge_index[1].astype(jnp.int32)
    e_tile = NB * CHUNK
    src2 = jnp.zeros((NW, NBP * CHUNK), jnp.int32)
    dst2 = jnp.full((NW, NBP * CHUNK), N_NODES, jnp.int32)
    src2 = src2.at[:, :e_tile].set(
        jnp.concatenate([src, jnp.zeros((NW * e_tile - N_EDGES,), jnp.int32)])
        .reshape(NW, e_tile))
    dst2 = dst2.at[:, :e_tile].set(
        jnp.concatenate([dst, jnp.full((NW * e_tile - N_EDGES,), N_NODES,
                                       jnp.int32)]).reshape(NW, e_tile))
    src3 = src2.reshape(NW, NBP, CHUNK)
    dst3 = dst2.reshape(NW, NBP, CHUNK)
    zeros = jnp.zeros((NPT, FEATS), jnp.float32)
    partial = _scatter_sc(src3, dst3, x, zeros)
    return _linear_tc(partial[0, :N_NODES], partial[1, :N_NODES],
                      W.T, b.reshape(1, FEATS))


# speedup vs baseline: 1.5665x; 1.3516x over previous
import functools

import jax
import jax.numpy as jnp
from jax import lax
from jax.experimental import pallas as pl
from jax.experimental.pallas import tpu as pltpu
from jax.experimental.pallas import tpu_sc as plsc

N_NODES = 10000
N_EDGES = 320000
FEATS = 128

NC = 2
NS = 16
NW = NC * NS
CHUNK = 128
NB = -(-N_EDGES // (NW * CHUNK))
E_PAD = NW * NB * CHUNK
NPT = (-(-N_NODES // NS) + 7) // 8 * 8
N_PAD = NPT * NS


def _scatter_body(src_hbm, dst_hbm, x_hbm, zeros_hbm, out_hbm,
                  src_v, dst_v, rows_v, acc_s, sem):
    cid = lax.axis_index("c")
    sid = lax.axis_index("s")
    wid = cid * NS + sid

    pltpu.sync_copy(zeros_hbm, acc_s.at[pl.ds(sid * NPT, NPT)])
    plsc.subcore_barrier()

    def body(j, carry):
        pltpu.sync_copy(src_hbm.at[wid, j], src_v)
        pltpu.async_copy(x_hbm.at[src_v], rows_v, sem).wait()
        pltpu.sync_copy(dst_hbm.at[wid, j], dst_v)
        pltpu.sync_copy(rows_v, acc_s.at[dst_v], add=True)
        return carry

    lax.fori_loop(0, NB, body, 0)
    plsc.subcore_barrier()

    pltpu.sync_copy(acc_s.at[pl.ds(sid * NPT, NPT)],
                    out_hbm.at[cid, pl.ds(sid * NPT, NPT)])


_scatter_sc = functools.partial(
    pl.kernel,
    mesh=plsc.VectorSubcoreMesh(core_axis_name="c", subcore_axis_name="s"),
    out_type=jax.ShapeDtypeStruct((NC, N_PAD, FEATS), jnp.float32),
    scratch_types=[
        pltpu.VMEM((CHUNK,), jnp.int32),
        pltpu.VMEM((CHUNK,), jnp.int32),
        pltpu.VMEM((CHUNK, FEATS), jnp.float32),
        pltpu.VMEM_SHARED((N_PAD, FEATS), jnp.float32),
        pltpu.SemaphoreType.DMA,
    ],
)(_scatter_body)


def _linear_body(p0_ref, p1_ref, wt_ref, b_ref, o_ref):
    h = p0_ref[...] + p1_ref[...]
    o_ref[...] = (
        jnp.dot(h, wt_ref[...], preferred_element_type=jnp.float32) + b_ref[...]
    )


def _linear_tc(p0, p1, wt, b2):
    m = p0.shape[0]
    bm = 1000
    return pl.pallas_call(
        _linear_body,
        grid=(m // bm,),
        in_specs=[
            pl.BlockSpec((bm, FEATS), lambda i: (i, 0)),
            pl.BlockSpec((bm, FEATS), lambda i: (i, 0)),
            pl.BlockSpec((FEATS, FEATS), lambda i: (0, 0)),
            pl.BlockSpec((1, FEATS), lambda i: (0, 0)),
        ],
        out_specs=pl.BlockSpec((bm, FEATS), lambda i: (i, 0)),
        out_shape=jax.ShapeDtypeStruct((m, FEATS), jnp.float32),
    )(p0, p1, wt, b2)


def kernel(x, edge_index, W, b):
    src = edge_index[0].astype(jnp.int32)
    dst = edge_index[1].astype(jnp.int32)
    pad = E_PAD - N_EDGES
    src = jnp.concatenate([src, jnp.zeros((pad,), jnp.int32)])
    dst = jnp.concatenate([dst, jnp.full((pad,), N_NODES, jnp.int32)])
    src3 = src.reshape(NW, NB, CHUNK)
    dst3 = dst.reshape(NW, NB, CHUNK)
    zeros = jnp.zeros((NPT, FEATS), jnp.float32)
    partial = _scatter_sc(src3, dst3, x, zeros)
    return _linear_tc(partial[0, :N_NODES], partial[1, :N_NODES],
                      W.T, b.reshape(1, FEATS))
